# baseline (speedup 1.0000x reference)
import functools

import jax
import jax.numpy as jnp
from jax import lax
from jax.experimental import pallas as pl
from jax.experimental.pallas import tpu as pltpu
from jax.experimental.pallas import tpu_sc as plsc

N = 10000
E = 320000
D = 128
NT = 100
NRBF = 16

NC = 2
NS = 16
NW = NC * NS
ACC = 238 * 128
BN = 2000
NBN = N // BN

EA = 163840
EB = E - EA

_mesh = plsc.VectorSubcoreMesh(core_axis_name="c", subcore_axis_name="s")
_sc_params = pltpu.CompilerParams(needs_layout_passes=False)


def _make_geom(ebase, ept):
    @functools.partial(
        pl.kernel,
        mesh=_mesh,
        compiler_params=_sc_params,
        out_type=[
            jax.ShapeDtypeStruct((NW * ept,), jnp.float32),
            jax.ShapeDtypeStruct((NW * ept,), jnp.float32),
            jax.ShapeDtypeStruct((NW * ept,), jnp.int32),
        ],
        scratch_types=[
            pltpu.VMEM((N,), jnp.float32),
            pltpu.VMEM((N,), jnp.float32),
            pltpu.VMEM((N,), jnp.float32),
            pltpu.VMEM((N,), jnp.int32),
            pltpu.VMEM((ept,), jnp.int32),
            pltpu.VMEM((ept,), jnp.int32),
            pltpu.VMEM((ept,), jnp.float32),
            pltpu.VMEM((ept,), jnp.float32),
            pltpu.VMEM((ept,), jnp.float32),
            pltpu.VMEM((ept,), jnp.float32),
            pltpu.VMEM((ept,), jnp.float32),
            pltpu.VMEM((ept,), jnp.int32),
            pltpu.VMEM((16,), jnp.float32),
            pltpu.SemaphoreType.DMA,
        ],
    )
    def geom(pxh, pyh, pzh, z_hbm, srch, dsth, sxh, syh, szh, w16,
             s_out, d_out, zs_out,
             px, py, pz, zn, sv, dv, sx, sy, sz, sb, db, zb, wv, sem):
        wid = lax.axis_index("s") * NC + lax.axis_index("c")
        gbase = ebase + wid * ept
        obase = wid * ept
        descs = [
            pltpu.async_copy(pxh, px, sem),
            pltpu.async_copy(pyh, py, sem),
            pltpu.async_copy(pzh, pz, sem),
            pltpu.async_copy(z_hbm, zn, sem),
            pltpu.async_copy(srch.at[pl.ds(gbase, ept)], sv, sem),
            pltpu.async_copy(dsth.at[pl.ds(gbase, ept)], dv, sem),
            pltpu.async_copy(sxh.at[pl.ds(gbase, ept)], sx, sem),
            pltpu.async_copy(syh.at[pl.ds(gbase, ept)], sy, sem),
            pltpu.async_copy(szh.at[pl.ds(gbase, ept)], sz, sem),
            pltpu.async_copy(w16, wv, sem),
        ]
        for de in descs:
            de.wait()
        one = jnp.full((16,), 1, jnp.int32)
        w1 = plsc.load_gather(wv, [one])
        w2 = plsc.load_gather(wv, [one + 1])
        w3 = plsc.load_gather(wv, [one + 2])

        @pl.loop(0, ept // 16, unroll=4)
        def body(i):
            off = i * 16
            s16 = sv[pl.ds(off, 16)]
            d16 = dv[pl.ds(off, 16)]
            ax = plsc.load_gather(px, [d16]) - plsc.load_gather(px, [s16]) + sx[pl.ds(off, 16)]
            ay = plsc.load_gather(py, [d16]) - plsc.load_gather(py, [s16]) + sy[pl.ds(off, 16)]
            az = plsc.load_gather(pz, [d16]) - plsc.load_gather(pz, [s16]) + sz[pl.ds(off, 16)]
            zg = plsc.load_gather(zn, [s16])
            sb[pl.ds(off, 16)] = ax * ax + ay * ay + az * az
            db[pl.ds(off, 16)] = ax * w1 + ay * w2 + az * w3
            zb[pl.ds(off, 16)] = zg

        outs = [
            pltpu.async_copy(sb, s_out.at[pl.ds(obase, ept)], sem),
            pltpu.async_copy(db, d_out.at[pl.ds(obase, ept)], sem),
            pltpu.async_copy(zb, zs_out.at[pl.ds(obase, ept)], sem),
        ]
        for de in outs:
            de.wait()

    return geom


_geom_a = _make_geom(0, EA // NW)
_geom_b = _make_geom(EA, EB // NW)


def _silu(x):
    return 0.5 * x + (0.5 * x) * jnp.tanh(0.5 * x)


def _make_edge(etot, be):
    nb = etot // be

    def edge_body(s_ref, d_ref, zs_ref, W3T_ref, embT_ref, b3c_ref,
                  W1cat_ref, b1_ref, cen_ref, w0_ref, W5T_ref, out_ref,
                  nfs_ref):
        i = pl.program_id(0)

        @pl.when(i == 0)
        def _():
            x = jnp.dot(W3T_ref[...], embT_ref[...],
                        preferred_element_type=jnp.float32) + b3c_ref[...]
            nfT = _silu(x)
            hi = nfT.astype(jnp.bfloat16)
            lo = (nfT - hi.astype(jnp.float32)).astype(jnp.bfloat16)
            nfs_ref[0:D, :] = hi
            nfs_ref[D:2 * D, :] = lo

        s = s_ref[0, :]
        dd = d_ref[0, :]
        zs = zs_ref[0, :]
        se = s + 1e-12
        inv = jax.lax.rsqrt(se)
        ln = se * inv
        shw = w0_ref[0, 0] + dd * inv
        diff = cen_ref[...] - ln[None, :]
        rbf = jnp.exp(-2.0 * diff * diff)
        rh = rbf.astype(jnp.bfloat16)
        rl = (rbf - rh.astype(jnp.float32)).astype(jnp.bfloat16)
        rcat = jnp.concatenate([rh, rl], axis=0)
        pre = jnp.dot(W1cat_ref[...], rcat, preferred_element_type=jnp.float32)
        pre = pre + b1_ref[...]
        ef = _silu(pre)
        ohT = (jax.lax.broadcasted_iota(jnp.int32, (D, be), 0) == zs[None, :])
        ohb = ohT.astype(jnp.bfloat16)
        sel = jnp.dot(nfs_ref[...], ohb, preferred_element_type=jnp.float32)
        nfT = sel[0:D, :] + sel[D:2 * D, :]
        t = nfT * ef * shw[None, :]
        out_ref[...] = jnp.dot(W5T_ref[...], t,
                               preferred_element_type=jnp.float32)

    return pl.pallas_call(
        edge_body,
        grid=(nb,),
        in_specs=[
            pl.BlockSpec((1, be), lambda i: (0, i)),
            pl.BlockSpec((1, be), lambda i: (0, i)),
            pl.BlockSpec((1, be), lambda i: (0, i)),
            pl.BlockSpec((D, D), lambda i: (0, 0)),
            pl.BlockSpec((D, D), lambda i: (0, 0)),
            pl.BlockSpec((D, 1), lambda i: (0, 0)),
            pl.BlockSpec((D, 2 * NRBF), lambda i: (0, 0)),
            pl.BlockSpec((D, 1), lambda i: (0, 0)),
            pl.BlockSpec((NRBF, 1), lambda i: (0, 0)),
            pl.BlockSpec((1, 1), lambda i: (0, 0)),
            pl.BlockSpec((3, D), lambda i: (0, 0)),
        ],
        out_specs=pl.BlockSpec((3, be), lambda i: (0, i)),
        out_shape=jax.ShapeDtypeStruct((3, etot), jnp.float32),
        scratch_shapes=[pltpu.VMEM((2 * D, D), jnp.bfloat16)],
    )


_edge_a = _make_edge(EA, 5120)
_edge_b = _make_edge(EB, 2560)


def _make_scat(ebase, ept):
    rows = (ept + 127) // 128
    pads = rows * 128 - ept
    full_rows = ept // 128

    @functools.partial(
        pl.kernel,
        mesh=_mesh,
        compiler_params=_sc_params,
        out_type=jax.ShapeDtypeStruct((NC * ACC,), jnp.float32),
        scratch_types=[
            pltpu.VMEM((ept,), jnp.int32),
            pltpu.VMEM((rows * 128,), jnp.float32),
            pltpu.VMEM((rows * 128,), jnp.float32),
            pltpu.VMEM((rows * 128,), jnp.float32),
            pltpu.VMEM((rows, 128), jnp.int32),
            pltpu.VMEM((rows, 128), jnp.int32),
            pltpu.VMEM((rows, 128), jnp.int32),
            pltpu.VMEM((ACC,), jnp.float32),
            pltpu.VMEM_SHARED((ACC,), jnp.float32),
            pltpu.SemaphoreType.DMA,
        ],
    )
    def scat(dsth, oxh, oyh, ozh, zeros_hbm, out_hbm,
             dv, ux, uy, uz, ix, iy, iz, bb, acc, sem):
        cid = lax.axis_index("c")
        sid = lax.axis_index("s")
        wid = sid * NC + cid
        gbase = ebase + wid * ept
        obase = wid * ept
        pltpu.sync_copy(dsth.at[pl.ds(gbase, ept)], dv)
        ins = [
            pltpu.async_copy(oxh.at[pl.ds(obase, ept)],
                             ux.at[pl.ds(0, ept)], sem),
            pltpu.async_copy(oyh.at[pl.ds(obase, ept)],
                             uy.at[pl.ds(0, ept)], sem),
            pltpu.async_copy(ozh.at[pl.ds(obase, ept)],
                             uz.at[pl.ds(0, ept)], sem),
        ]

        @pl.when(sid == 0)
        def _():
            pltpu.sync_copy(zeros_hbm, bb)
            pltpu.sync_copy(bb, acc)

        @pl.loop(0, full_rows, unroll=4)
        def ibody(r):
            for k in range(8):
                d16 = dv[pl.ds(r * 128 + k * 16, 16)]
                f = d16 * 3
                ix[r, pl.ds(k * 16, 16)] = f
                iy[r, pl.ds(k * 16, 16)] = f + 1
                iz[r, pl.ds(k * 16, 16)] = f + 2

        if pads:
            d16 = dv[pl.ds(full_rows * 128, 16)]
            f = d16 * 3
            ix[rows - 1, pl.ds(0, 16)] = f
            iy[rows - 1, pl.ds(0, 16)] = f + 1
            iz[rows - 1, pl.ds(0, 16)] = f + 2
            ar16 = jnp.arange(16, dtype=jnp.int32)
            for k in range(1, 8):
                pad = 3 * N + (k - 1) * 16 + ar16
                ix[rows - 1, pl.ds(k * 16, 16)] = pad
                iy[rows - 1, pl.ds(k * 16, 16)] = pad
                iz[rows - 1, pl.ds(k * 16, 16)] = pad

        for de in ins:
            de.wait()
        if pads:
            zero16 = jnp.zeros((16,), jnp.float32)
            for k in range(pads // 16):
                off = ept + k * 16
                ux[pl.ds(off, 16)] = zero16
                uy[pl.ds(off, 16)] = zero16
                uz[pl.ds(off, 16)] = zero16

        plsc.subcore_barrier()

        def fire(r):
            descs = []
            for u, ii in ((ux, ix), (uy, iy), (uz, iz)):
                descs.append(pltpu.async_copy(
                    u.at[pl.ds(r * 128, 128)], acc.at[ii.at[r]], sem,
                    add=True))
            return descs

        @pl.loop(0, rows // 8)
        def sbody(g):
            descs = []
            for rr in range(8):
                descs.extend(fire(g * 8 + rr))
            for de in descs:
                de.wait()

        if rows % 8:
            descs = []
            for rr in range(rows % 8):
                descs.extend(fire((rows // 8) * 8 + rr))
            for de in descs:
                de.wait()

        plsc.subcore_barrier()

        @pl.when(sid == 0)
        def _():
            pltpu.sync_copy(acc, bb)
            pltpu.sync_copy(bb, out_hbm.at[pl.ds(cid * ACC, ACC)])

    return scat


_scat_a = _make_scat(0, EA // NW)
_scat_b = _make_scat(EA, EB // NW)


def _fin_body(pa_ref, pb_ref, z_ref, emb_ref, W4_ref, W5_ref, b5_ref,
              o_ref, w45_ref):
    i = pl.program_id(0)

    @pl.when(i == 0)
    def _():
        w45 = jnp.dot(W4_ref[...], W5_ref[...],
                      preferred_element_type=jnp.float32)
        w45_ref[...] = jnp.dot(emb_ref[...], w45,
                               preferred_element_type=jnp.float32)

    zb = z_ref[0, 0, :]
    oh = (jax.lax.broadcasted_iota(jnp.int32, (BN, D), 1) == zb[:, None])
    bse = jnp.dot(oh.astype(jnp.float32), w45_ref[...],
                  preferred_element_type=jnp.float32)
    o_ref[...] = (pa_ref[0] + pa_ref[1] + pb_ref[0] + pb_ref[1]
                  + bse + b5_ref[...][None, :])


_fin = pl.pallas_call(
    _fin_body,
    grid=(NBN,),
    in_specs=[
        pl.BlockSpec((2, BN, 3), lambda i: (0, i, 0)),
        pl.BlockSpec((2, BN, 3), lambda i: (0, i, 0)),
        pl.BlockSpec((1, 1, BN), lambda i: (i, 0, 0)),
        pl.BlockSpec((D, D), lambda i: (0, 0)),
        pl.BlockSpec((D, D), lambda i: (0, 0)),
        pl.BlockSpec((D, 3), lambda i: (0, 0)),
        pl.BlockSpec((3,), lambda i: (0,)),
    ],
    out_specs=pl.BlockSpec((BN, 3), lambda i: (i, 0)),
    out_shape=jax.ShapeDtypeStruct((N, 3), jnp.float32),
    scratch_shapes=[pltpu.VMEM((D, 3), jnp.float32)],
)


def kernel(z, pos, edge_index, shift_vector, emb, rbf_centers, W1, b1,
           W3, b3, w_sh, W4, W5, b5):
    f32 = jnp.float32
    bf16 = jnp.bfloat16
    w16 = jnp.pad(w_sh, (0, 12))
    emb128 = jnp.pad(emb, ((0, D - NT), (0, 0)))
    W1T = W1.T
    W1h = W1T.astype(bf16)
    W1l = (W1T - W1h.astype(f32)).astype(bf16)
    W1cat = jnp.concatenate([W1h, W1l], axis=1)
    px, py, pz = pos[:, 0], pos[:, 1], pos[:, 2]
    shx, shy, shz = shift_vector[:, 0], shift_vector[:, 1], shift_vector[:, 2]
    src, dst = edge_index[0], edge_index[1]
    zeros = jnp.zeros((ACC,), f32)
    wargs_b = (W3.T, emb128.T, b3.reshape(D, 1), W1cat, b1.reshape(D, 1),
               rbf_centers.reshape(NRBF, 1), w_sh[0].reshape(1, 1), W5.T)

    sa, da_, zsa = _geom_a(px, py, pz, z, src, dst, shx, shy, shz, w16)
    sb_, db_, zsb = _geom_b(px, py, pz, z, src, dst, shx, shy, shz, w16)
    o3a = _edge_a(sa.reshape(1, EA), da_.reshape(1, EA), zsa.reshape(1, EA),
                  *wargs_b)
    o3b = _edge_b(sb_.reshape(1, EB), db_.reshape(1, EB), zsb.reshape(1, EB),
                  *wargs_b)
    parts_a = _scat_a(dst, o3a[0], o3a[1], o3a[2], zeros)
    parts_b = _scat_b(dst, o3b[0], o3b[1], o3b[2], zeros)
    pa = parts_a.reshape(NC, ACC)[:, :3 * N].reshape(NC, N, 3)
    pb = parts_b.reshape(NC, ACC)[:, :3 * N].reshape(NC, N, 3)
    return _fin(pa, pb, z.reshape(NBN, 1, BN), emb128, W4, W5, b5)
